# baseline (speedup 1.0000x reference)
import functools

import jax
import jax.numpy as jnp
from jax import lax
from jax.experimental import pallas as pl
from jax.experimental.pallas import tpu as pltpu


def _bilstm_l1_kernel(gxf_ref, gxb_ref, len_ref, whh_ref,
                      outf_ref, outb_ref,
                      hf_sc, cf_sc, hb_sc, cb_sc, *, tt, hidden):
    ci = pl.program_id(1)
    nc = pl.num_programs(1)

    @pl.when(ci == 0)
    def _():
        hf_sc[...] = jnp.zeros_like(hf_sc)
        cf_sc[...] = jnp.zeros_like(cf_sc)
        hb_sc[...] = jnp.zeros_like(hb_sc)
        cb_sc[...] = jnp.zeros_like(cb_sc)

    w_f = whh_ref[0]
    w_b = whh_ref[1]
    lens = len_ref[...]

    t0_f = ci * tt
    t0_b = (nc - 1 - ci) * tt

    def cell(gx_t, h, c_prev, w_hh, t):
        gates = gx_t + jnp.dot(h, w_hh, preferred_element_type=jnp.float32)
        i_g = jax.nn.sigmoid(gates[:, 0 * hidden:1 * hidden])
        f_g = jax.nn.sigmoid(gates[:, 1 * hidden:2 * hidden])
        g_g = jnp.tanh(gates[:, 2 * hidden:3 * hidden])
        o_g = jax.nn.sigmoid(gates[:, 3 * hidden:4 * hidden])
        c_new = f_g * c_prev + i_g * g_g
        h_new = (o_g * jnp.tanh(c_new)).astype(h.dtype)
        live = lens > t
        return jnp.where(live, h_new, h), jnp.where(live, c_new, c_prev)

    def step(s, carry):
        h_f, c_f, h_b, c_b = carry
        sb = tt - 1 - s
        h_f, c_f = cell(gxf_ref[s].astype(jnp.float32), h_f, c_f, w_f,
                        t0_f + s)
        h_b, c_b = cell(gxb_ref[sb].astype(jnp.float32), h_b, c_b, w_b,
                        t0_b + sb)
        outf_ref[s] = h_f
        outb_ref[sb] = h_b
        return h_f, c_f, h_b, c_b

    carry0 = (hf_sc[...], cf_sc[...], hb_sc[...], cb_sc[...])
    h_f, c_f, h_b, c_b = lax.fori_loop(0, tt, step, carry0, unroll=True)

    hf_sc[...] = h_f
    cf_sc[...] = c_f
    hb_sc[...] = h_b
    cb_sc[...] = c_b


def _bilstm_l2_kernel(gxb_ref, len_ref, whh_ref,
                      gamma_ref, beta_ref, mean_ref, var_ref, w_ref, bias_ref,
                      out_ref,
                      hb_sc, cb_sc, *, tt, hidden):
    ci = pl.program_id(1)
    nc = pl.num_programs(1)

    @pl.when(ci == 0)
    def _():
        hb_sc[...] = jnp.zeros_like(hb_sc)
        cb_sc[...] = jnp.zeros_like(cb_sc)

    w_b = whh_ref[...]
    lens = len_ref[...]
    t0_b = (nc - 1 - ci) * tt

    def step(s, carry):
        h_b, c_b = carry
        sb = tt - 1 - s
        gates = (gxb_ref[sb].astype(jnp.float32)
                 + jnp.dot(h_b, w_b, preferred_element_type=jnp.float32))
        i_g = jax.nn.sigmoid(gates[:, 0 * hidden:1 * hidden])
        f_g = jax.nn.sigmoid(gates[:, 1 * hidden:2 * hidden])
        g_g = jnp.tanh(gates[:, 2 * hidden:3 * hidden])
        o_g = jax.nn.sigmoid(gates[:, 3 * hidden:4 * hidden])
        c_new = f_g * c_b + i_g * g_g
        h_new = (o_g * jnp.tanh(c_new)).astype(h_b.dtype)
        live = lens > (t0_b + sb)
        return jnp.where(live, h_new, h_b), jnp.where(live, c_new, c_b)

    h_b, c_b = lax.fori_loop(0, tt, step, (hb_sc[...], cb_sc[...]),
                             unroll=True)
    hb_sc[...] = h_b
    cb_sc[...] = c_b

    @pl.when(ci == nc - 1)
    def _():
        x = h_b.astype(jnp.float32)
        inv_std = lax.rsqrt(var_ref[...] + 1e-5)
        y = jnp.tanh((x - mean_ref[...]) * inv_std * gamma_ref[...]
                     + beta_ref[...])
        out_ref[...] = (jnp.dot(y, w_ref[...],
                                preferred_element_type=jnp.float32)
                        + bias_ref[...])


def _input_proj(streams, w_ih, bias, mm_dtype):
    acc = None
    off = 0
    for s in streams:
        d = s.shape[-1]
        w = w_ih[off:off + d].astype(mm_dtype)
        p = jnp.einsum('tbd,dg->tbg', s.astype(mm_dtype), w,
                       preferred_element_type=jnp.float32)
        acc = p if acc is None else acc + p
        off += d
    return (acc + bias.astype(jnp.float32)).astype(mm_dtype)


def kernel(x, lengths, l0_wihf, l0_whhf, l0_bf, l0_wihb, l0_whhb, l0_bb,
           l1_wihf, l1_whhf, l1_bf, l1_wihb, l1_whhb, l1_bb,
           bn_gamma, bn_beta, bn_mean, bn_var, fc_w, fc_b,
           *, time_chunk=32, matmul_dtype=jnp.bfloat16):
    xt = jnp.transpose(x, (1, 0, 2)).astype(jnp.float32)
    T, B, _ = xt.shape
    H = l0_whhf.shape[0]
    G = fc_w.shape[1]

    lens32 = lengths.astype(jnp.int32)
    B_pad = ((B + 7) // 8) * 8
    if B_pad != B:
        xt = jnp.pad(xt, ((0, 0), (0, B_pad - B), (0, 0)))
        lens32 = jnp.pad(lens32, (0, B_pad - B))
    lengths_b1 = lens32[:, None]

    tt = int(min(time_chunk, T))
    tp = ((T + tt - 1) // tt) * tt
    if tp != T:
        xt = jnp.pad(xt, ((0, tp - T), (0, 0), (0, 0)))
    nc = tp // tt

    nb = 2 if B_pad % 16 == 0 else 1
    Bh = B_pad // nb
    msz = jnp.dtype(matmul_dtype).itemsize

    gx_f = _input_proj([xt], l0_wihf, l0_bf, matmul_dtype)
    gx_b = _input_proj([xt], l0_wihb, l0_bb, matmul_dtype)
    whh = jnp.stack([l0_whhf, l0_whhb]).astype(matmul_dtype)

    four_h = 4 * H
    need1 = (4 * tt * Bh * four_h * msz
             + 4 * tt * Bh * H * msz
             + 4 * H * four_h * msz
             + 2 * Bh * 4
             + 2 * Bh * H * (msz + 4))
    vmem1 = int(min(56 * 2**20, max(32 * 2**20, 1.5 * need1)))

    out_f, out_b = pl.pallas_call(
        functools.partial(_bilstm_l1_kernel, tt=tt, hidden=H),
        out_shape=(jax.ShapeDtypeStruct((tp, B_pad, H), matmul_dtype),
                   jax.ShapeDtypeStruct((tp, B_pad, H), matmul_dtype)),
        grid=(nb, nc),
        in_specs=[
            pl.BlockSpec((tt, Bh, four_h), lambda m, c: (c, m, 0)),
            pl.BlockSpec((tt, Bh, four_h), lambda m, c, nc=nc: (nc - 1 - c, m, 0)),
            pl.BlockSpec((Bh, 1), lambda m, c: (m, 0)),
            pl.BlockSpec((2, H, four_h), lambda m, c: (0, 0, 0)),
        ],
        out_specs=(
            pl.BlockSpec((tt, Bh, H), lambda m, c: (c, m, 0)),
            pl.BlockSpec((tt, Bh, H), lambda m, c, nc=nc: (nc - 1 - c, m, 0)),
        ),
        scratch_shapes=[pltpu.VMEM((Bh, H), matmul_dtype),
                        pltpu.VMEM((Bh, H), jnp.float32),
                        pltpu.VMEM((Bh, H), matmul_dtype),
                        pltpu.VMEM((Bh, H), jnp.float32)],
        compiler_params=pltpu.CompilerParams(
            dimension_semantics=("parallel", "arbitrary"),
            vmem_limit_bytes=vmem1),
    )(gx_f, gx_b, lengths_b1, whh)

    gx_b2 = _input_proj([out_f, out_b], l1_wihb, l1_bb, matmul_dtype)
    whh_b2 = l1_whhb.astype(matmul_dtype)

    need2 = (2 * tt * Bh * four_h * msz
             + 2 * H * four_h * msz
             + 2 * Bh * G * 4
             + 2 * Bh * 4
             + Bh * H * (msz + 4))
    vmem2 = int(min(56 * 2**20, max(32 * 2**20, 1.5 * need2)))

    out = pl.pallas_call(
        functools.partial(_bilstm_l2_kernel, tt=tt, hidden=H),
        out_shape=jax.ShapeDtypeStruct((B_pad, G), jnp.float32),
        grid=(nb, nc),
        in_specs=[
            pl.BlockSpec((tt, Bh, four_h), lambda m, c, nc=nc: (nc - 1 - c, m, 0)),
            pl.BlockSpec((Bh, 1), lambda m, c: (m, 0)),
            pl.BlockSpec((H, four_h), lambda m, c: (0, 0)),
            pl.BlockSpec((1, H), lambda m, c: (0, 0)),
            pl.BlockSpec((1, H), lambda m, c: (0, 0)),
            pl.BlockSpec((1, H), lambda m, c: (0, 0)),
            pl.BlockSpec((1, H), lambda m, c: (0, 0)),
            pl.BlockSpec((H, G), lambda m, c: (0, 0)),
            pl.BlockSpec((1, G), lambda m, c: (0, 0)),
        ],
        out_specs=pl.BlockSpec((Bh, G), lambda m, c: (m, 0)),
        scratch_shapes=[pltpu.VMEM((Bh, H), matmul_dtype),
                        pltpu.VMEM((Bh, H), jnp.float32)],
        compiler_params=pltpu.CompilerParams(
            dimension_semantics=("parallel", "arbitrary"),
            vmem_limit_bytes=vmem2),
    )(gx_b2, lengths_b1, whh_b2, bn_gamma, bn_beta, bn_mean, bn_var,
      fc_w, fc_b)

    return out[:B]


# speedup vs baseline: 1.8214x; 1.8214x over previous
import functools

import jax
import jax.numpy as jnp
from jax import lax
from jax.experimental import pallas as pl
from jax.experimental.pallas import tpu as pltpu


def _proj(x_ref, w_ref, b_ref, *, tt, B):
    d = x_ref.shape[-1]
    xc = x_ref[...].reshape(tt * B, d)
    acc = jnp.dot(xc, w_ref[...], preferred_element_type=jnp.float32)
    acc = acc + b_ref[...].astype(jnp.float32)
    return acc.astype(x_ref.dtype).reshape(tt, B, acc.shape[-1])


def _lstm_cell(gx_t, h, c_prev, w_hh, lens, t, hidden):
    gates = gx_t.astype(jnp.float32) + jnp.dot(
        h, w_hh, preferred_element_type=jnp.float32)
    i_g = jax.nn.sigmoid(gates[:, 0 * hidden:1 * hidden])
    f_g = jax.nn.sigmoid(gates[:, 1 * hidden:2 * hidden])
    g_g = jnp.tanh(gates[:, 2 * hidden:3 * hidden])
    o_g = jax.nn.sigmoid(gates[:, 3 * hidden:4 * hidden])
    c_new = f_g * c_prev + i_g * g_g
    h_new = (o_g * jnp.tanh(c_new)).astype(h.dtype)
    live = lens > t
    return jnp.where(live, h_new, h), jnp.where(live, c_new, c_prev)


def _l1_kernel(xf_ref, xb_ref, wf_ref, wb_ref,
               bf_ref, bb_ref, len_ref, whh_ref,
               outf_ref, outb_ref,
               gxf_sc, gxb_sc, hf_sc, cf_sc, hb_sc, cb_sc, *, tt, hidden):
    ci = pl.program_id(0)
    nc = pl.num_programs(0)
    B = len_ref.shape[0]

    @pl.when(ci == 0)
    def _():
        hf_sc[...] = jnp.zeros_like(hf_sc)
        cf_sc[...] = jnp.zeros_like(cf_sc)
        hb_sc[...] = jnp.zeros_like(hb_sc)
        cb_sc[...] = jnp.zeros_like(cb_sc)

    gxf_sc[...] = _proj(xf_ref, wf_ref, bf_ref, tt=tt, B=B)
    gxb_sc[...] = _proj(xb_ref, wb_ref, bb_ref, tt=tt, B=B)

    w_f = whh_ref[0]
    w_b = whh_ref[1]
    lens = len_ref[...]
    t0_f = ci * tt
    t0_b = (nc - 1 - ci) * tt

    def step(s, carry):
        h_f, c_f, h_b, c_b = carry
        sb = tt - 1 - s
        h_f, c_f = _lstm_cell(gxf_sc[s], h_f, c_f, w_f, lens,
                              t0_f + s, hidden)
        h_b, c_b = _lstm_cell(gxb_sc[sb], h_b, c_b, w_b, lens,
                              t0_b + sb, hidden)
        outf_ref[s] = h_f
        outb_ref[sb] = h_b
        return h_f, c_f, h_b, c_b

    carry0 = (hf_sc[...], cf_sc[...], hb_sc[...], cb_sc[...])
    h_f, c_f, h_b, c_b = lax.fori_loop(0, tt, step, carry0, unroll=True)

    hf_sc[...] = h_f
    cf_sc[...] = c_f
    hb_sc[...] = h_b
    cb_sc[...] = c_b


def _l2_kernel(f_ref, b_ref, wtop_ref, wbot_ref, bias2_ref,
               len_ref, whh_ref,
               gamma_ref, beta_ref, mean_ref, var_ref, wfc_ref, bfc_ref,
               out_ref,
               gx_sc, hb_sc, cb_sc, *, tt, hidden):
    ci = pl.program_id(0)
    nc = pl.num_programs(0)
    B = len_ref.shape[0]

    def proj2(f_ref, b_ref):
        fc = f_ref[...].reshape(tt * B, hidden)
        bc = b_ref[...].reshape(tt * B, hidden)
        acc = jnp.dot(fc, wtop_ref[...], preferred_element_type=jnp.float32)
        acc = acc + jnp.dot(bc, wbot_ref[...],
                            preferred_element_type=jnp.float32)
        acc = acc + bias2_ref[...].astype(jnp.float32)
        return acc.astype(f_ref.dtype).reshape(tt, B, 4 * hidden)

    @pl.when(ci == 0)
    def _():
        hb_sc[...] = jnp.zeros_like(hb_sc)
        cb_sc[...] = jnp.zeros_like(cb_sc)

    gx_sc[...] = proj2(f_ref, b_ref)

    w_b = whh_ref[...]
    lens = len_ref[...]
    t0_b = (nc - 1 - ci) * tt

    def step(s, carry):
        h_b, c_b = carry
        sb = tt - 1 - s
        return _lstm_cell(gx_sc[sb], h_b, c_b, w_b, lens,
                          t0_b + sb, hidden)

    h_b, c_b = lax.fori_loop(0, tt, step, (hb_sc[...], cb_sc[...]),
                             unroll=True)
    hb_sc[...] = h_b
    cb_sc[...] = c_b

    @pl.when(ci == nc - 1)
    def _():
        x = h_b.astype(jnp.float32)
        inv_std = lax.rsqrt(var_ref[...] + 1e-5)
        y = jnp.tanh((x - mean_ref[...]) * inv_std * gamma_ref[...]
                     + beta_ref[...])
        out_ref[...] = (jnp.dot(y, wfc_ref[...],
                                preferred_element_type=jnp.float32)
                        + bfc_ref[...])


def kernel(x, lengths, l0_wihf, l0_whhf, l0_bf, l0_wihb, l0_whhb, l0_bb,
           l1_wihf, l1_whhf, l1_bf, l1_wihb, l1_whhb, l1_bb,
           bn_gamma, bn_beta, bn_mean, bn_var, fc_w, fc_b,
           *, time_chunk=32, matmul_dtype=jnp.bfloat16):
    B, T, D = x.shape
    H = l0_whhf.shape[0]
    G = fc_w.shape[1]
    four_h = 4 * H

    xt = jnp.transpose(x.astype(matmul_dtype), (1, 0, 2))

    lens32 = lengths.astype(jnp.int32)
    B_pad = ((B + 7) // 8) * 8
    if B_pad != B:
        xt = jnp.pad(xt, ((0, 0), (0, B_pad - B), (0, 0)))
        lens32 = jnp.pad(lens32, (0, B_pad - B))
    lengths_b1 = lens32[:, None]

    tt = int(min(time_chunk, T))
    tp = ((T + tt - 1) // tt) * tt
    if tp != T:
        xt = jnp.pad(xt, ((0, tp - T), (0, 0), (0, 0)))
    nc = tp // tt

    msz = jnp.dtype(matmul_dtype).itemsize
    whh = jnp.stack([l0_whhf, l0_whhb]).astype(matmul_dtype)
    wf = l0_wihf.astype(matmul_dtype)
    wb = l0_wihb.astype(matmul_dtype)

    need1 = (6 * tt * B_pad * D * msz
             + 2 * D * four_h * msz
             + 2 * H * four_h * msz * 2
             + 4 * tt * B_pad * four_h * msz
             + 4 * tt * B_pad * H * msz
             + 2 * B_pad * H * (msz + 4)
             + 4 * B_pad)
    vmem1 = int(min(60 * 2**20, max(32 * 2**20, 1.3 * need1)))

    out_f, out_b = pl.pallas_call(
        functools.partial(_l1_kernel, tt=tt, hidden=H),
        out_shape=(jax.ShapeDtypeStruct((tp, B_pad, H), matmul_dtype),
                   jax.ShapeDtypeStruct((tp, B_pad, H), matmul_dtype)),
        grid=(nc,),
        in_specs=[
            pl.BlockSpec((tt, B_pad, D), lambda c: (c, 0, 0)),
            pl.BlockSpec((tt, B_pad, D), lambda c, nc=nc: (nc - 1 - c, 0, 0)),
            pl.BlockSpec((D, four_h), lambda c: (0, 0)),
            pl.BlockSpec((D, four_h), lambda c: (0, 0)),
            pl.BlockSpec((1, four_h), lambda c: (0, 0)),
            pl.BlockSpec((1, four_h), lambda c: (0, 0)),
            pl.BlockSpec((B_pad, 1), lambda c: (0, 0)),
            pl.BlockSpec((2, H, four_h), lambda c: (0, 0, 0)),
        ],
        out_specs=(
            pl.BlockSpec((tt, B_pad, H), lambda c: (c, 0, 0)),
            pl.BlockSpec((tt, B_pad, H), lambda c, nc=nc: (nc - 1 - c, 0, 0)),
        ),
        scratch_shapes=[pltpu.VMEM((tt, B_pad, four_h), matmul_dtype),
                        pltpu.VMEM((tt, B_pad, four_h), matmul_dtype),
                        pltpu.VMEM((B_pad, H), matmul_dtype),
                        pltpu.VMEM((B_pad, H), jnp.float32),
                        pltpu.VMEM((B_pad, H), matmul_dtype),
                        pltpu.VMEM((B_pad, H), jnp.float32)],
        compiler_params=pltpu.CompilerParams(
            dimension_semantics=("arbitrary",),
            vmem_limit_bytes=vmem1),
    )(xt, xt, wf, wb, l0_bf, l0_bb, lengths_b1, whh)

    wtop = l1_wihb[:H].astype(matmul_dtype)
    wbot = l1_wihb[H:].astype(matmul_dtype)
    whh_b2 = l1_whhb.astype(matmul_dtype)

    need2 = (12 * tt * B_pad * H * msz
             + 2 * H * four_h * msz
             + H * four_h * msz * 2
             + 2 * tt * B_pad * four_h * msz
             + B_pad * H * (msz + 4)
             + (H * G + 2 * B_pad * G) * 4
             + 4 * B_pad)
    vmem2 = int(min(60 * 2**20, max(32 * 2**20, 1.3 * need2)))

    out = pl.pallas_call(
        functools.partial(_l2_kernel, tt=tt, hidden=H),
        out_shape=jax.ShapeDtypeStruct((B_pad, G), jnp.float32),
        grid=(nc,),
        in_specs=[
            pl.BlockSpec((tt, B_pad, H), lambda c, nc=nc: (nc - 1 - c, 0, 0)),
            pl.BlockSpec((tt, B_pad, H), lambda c, nc=nc: (nc - 1 - c, 0, 0)),
            pl.BlockSpec((H, four_h), lambda c: (0, 0)),
            pl.BlockSpec((H, four_h), lambda c: (0, 0)),
            pl.BlockSpec((1, four_h), lambda c: (0, 0)),
            pl.BlockSpec((B_pad, 1), lambda c: (0, 0)),
            pl.BlockSpec((H, four_h), lambda c: (0, 0)),
            pl.BlockSpec((1, H), lambda c: (0, 0)),
            pl.BlockSpec((1, H), lambda c: (0, 0)),
            pl.BlockSpec((1, H), lambda c: (0, 0)),
            pl.BlockSpec((1, H), lambda c: (0, 0)),
            pl.BlockSpec((H, G), lambda c: (0, 0)),
            pl.BlockSpec((1, G), lambda c: (0, 0)),
        ],
        out_specs=pl.BlockSpec((B_pad, G), lambda c: (0, 0)),
        scratch_shapes=[pltpu.VMEM((tt, B_pad, four_h), matmul_dtype),
                        pltpu.VMEM((B_pad, H), matmul_dtype),
                        pltpu.VMEM((B_pad, H), jnp.float32)],
        compiler_params=pltpu.CompilerParams(
            dimension_semantics=("arbitrary",),
            vmem_limit_bytes=vmem2),
    )(out_f, out_b, wtop, wbot, l1_bb, lengths_b1, whh_b2,
      bn_gamma, bn_beta, bn_mean, bn_var, fc_w, fc_b)

    return out[:B]
